# baseline (speedup 1.0000x reference)
import functools

import jax
import jax.numpy as jnp
from jax.experimental import pallas as pl
from jax.experimental.pallas import tpu as pltpu

B, N, D, E = 16, 1000, 2, 128
K = 10
PB = 2
HIGH = jax.lax.Precision.HIGHEST


def _fused_kernel(loc_ref, locT_ref, w_init_ref, b_init_ref, w_nbr_ref,
                  b_nbr_ref, w_fin_ref, b_fin_ref, depot_ref, w_dep_ref,
                  b_dep_ref, bn_w_ref, bn_b_ref,
                  h_ref, hmean_ref,
                  fe_s, a_s, phi_s, plo_s, c_s, dep_s, stats_s):
    p = pl.program_id(0)
    b = pl.program_id(1)

    @pl.when(jnp.logical_and(p == 0, b == 0))
    def _prep():
        w_fin = w_fin_ref[...]
        a_s[...] = jax.lax.dot_general(
            w_init_ref[...] - 10.0 * w_nbr_ref[...], w_fin,
            (((1,), (0,)), ((), ())), precision=HIGH,
            preferred_element_type=jnp.float32)
        bw = jax.lax.dot_general(
            w_nbr_ref[...], w_fin, (((1,), (0,)), ((), ())),
            precision=HIGH, preferred_element_type=jnp.float32)
        pm = jax.lax.dot_general(
            loc_ref[0], bw, (((1,), (0,)), ((), ())),
            precision=HIGH, preferred_element_type=jnp.float32)
        phi = pm.astype(jnp.bfloat16)
        phi_s[...] = phi
        plo_s[...] = (pm - phi.astype(jnp.float32)).astype(jnp.bfloat16)
        c_s[...] = jax.lax.dot_general(
            b_init_ref[...] + 10.0 * b_nbr_ref[...], w_fin,
            (((1,), (0,)), ((), ())), precision=HIGH,
            preferred_element_type=jnp.float32) + 11.0 * b_fin_ref[...]
        dep_s[...] = jax.lax.dot_general(
            depot_ref[...], w_dep_ref[...], (((1,), (0,)), ((), ())),
            precision=HIGH,
            preferred_element_type=jnp.float32) + b_dep_ref[...]

    @pl.when(p == 0)
    def _main():
        row_i = jax.lax.broadcasted_iota(jnp.int32, (N, N), 0)
        col_i = jax.lax.broadcasted_iota(jnp.int32, (N, N), 1)
        diag = row_i == col_i

        def _dist_of(t):
            xq = loc_ref[t]
            xqT = locT_ref[t]
            dx = xq[:, 0:1] - xqT[0:1, :]
            dy = xq[:, 1:2] - xqT[1:2, :]
            return jnp.sqrt(dx * dx + dy * dy)

        d0a = _dist_of(0)
        d0b = _dist_of(1)
        da = jnp.where(diag, jnp.inf, d0a)
        db = jnp.where(diag, jnp.inf, d0b)
        for _ in range(K - 1):
            ma = jnp.min(da, axis=0, keepdims=True)
            mb = jnp.min(db, axis=0, keepdims=True)
            da = jnp.where(da == ma, jnp.inf, da)
            db = jnp.where(db == mb, jnp.inf, db)

        def _fe_of(xq, mk):
            return (jax.lax.dot_general(xq, a_s[...],
                                        (((1,), (0,)), ((), ())),
                                        precision=HIGH,
                                        preferred_element_type=jnp.float32)
                    + jax.lax.dot_general(mk, phi_s[...],
                                          (((0,), (0,)), ((), ())),
                                          preferred_element_type=jnp.float32)
                    + jax.lax.dot_general(mk, plo_s[...],
                                          (((0,), (0,)), ((), ())),
                                          preferred_element_type=jnp.float32)
                    + c_s[...])

        part = jnp.zeros((2, E), jnp.float32)
        for t, (d, d0) in enumerate(((da, d0a), (db, d0b))):
            marked = d == jnp.inf
            mask = marked.astype(jnp.bfloat16)
            cnt = jnp.sum(marked.astype(jnp.float32), axis=0,
                          keepdims=True)
            fe_s[PB * b + t] = _fe_of(loc_ref[t], mask)

            @pl.when(jnp.max(cnt) > jnp.float32(K))
            def _exact_fallback(t=t, d0=d0):
                iota_f = row_i.astype(jnp.float32)
                big = jnp.float32(2e9)
                d2 = jnp.where(diag, jnp.inf, d0)
                for _ in range(K - 1):
                    m2 = jnp.min(d2, axis=0, keepdims=True)
                    idx = jnp.min(jnp.where(d2 == m2, iota_f, big),
                                  axis=0, keepdims=True)
                    d2 = jnp.where(iota_f == idx, jnp.inf, d2)
                fe_s[PB * b + t] = _fe_of(
                    loc_ref[t], (d2 == jnp.inf).astype(jnp.bfloat16))

            fe = fe_s[PB * b + t]
            part = part + jnp.stack([jnp.sum(fe, axis=0),
                                     jnp.sum(fe * fe, axis=0)])

        @pl.when(b == 0)
        def _():
            stats_s[...] = part

        @pl.when(b != 0)
        def _():
            stats_s[...] += part

    @pl.when(p == 1)
    def _finish():
        n_rows = jnp.float32(B * N)
        mean = stats_s[0:1, :] / n_rows
        var = stats_s[1:2, :] / n_rows - mean * mean
        inv = jax.lax.rsqrt(var + 1e-5)
        scale = inv * bn_w_ref[...]
        shift = bn_b_ref[...] - mean * scale

        for t in range(PB):
            fe = fe_s[PB * b + t]
            normed = fe * scale + shift
            h = jnp.where(normed >= 0, normed, 0.01 * normed)

            dep = dep_s[pl.ds(PB * b + t, 1), :]
            hdep = jnp.where(dep >= 0, dep, 0.01 * dep)
            h_ref[t, 0:1, :] = hdep
            h_ref[t, pl.ds(1, N), :] = h
            hmean_ref[t] = (hdep + jnp.sum(h, axis=0, keepdims=True)) * (
                1.0 / jnp.float32(N + 1))


@functools.partial(jax.jit, static_argnames=())
def kernel(loc, depot, W_init, b_init, W_nbr, b_nbr, W_fin, b_fin,
           W_dep, b_dep, bn_w, bn_b):
    f32 = jnp.float32
    locT = jnp.swapaxes(loc, 1, 2)
    depot2 = depot.reshape(B, 2)
    b_init2 = b_init.reshape(1, -1)
    b_nbr2 = b_nbr.reshape(1, -1)
    b_fin2 = b_fin.reshape(1, -1)
    b_dep2 = b_dep.reshape(1, -1)
    bn_w2 = bn_w.reshape(1, -1)
    bn_b2 = bn_b.reshape(1, -1)

    const = lambda p, b: (0, 0)
    h, h_mean = pl.pallas_call(
        _fused_kernel,
        grid=(2, B // PB),
        in_specs=[
            pl.BlockSpec((PB, N, D), lambda p, b: (b, 0, 0)),
            pl.BlockSpec((PB, D, N), lambda p, b: (b, 0, 0)),
            pl.BlockSpec((D, 2 * E), const),
            pl.BlockSpec((1, 2 * E), const),
            pl.BlockSpec((D, 2 * E), const),
            pl.BlockSpec((1, 2 * E), const),
            pl.BlockSpec((2 * E, E), const),
            pl.BlockSpec((1, E), const),
            pl.BlockSpec((B, D), const),
            pl.BlockSpec((D, E), const),
            pl.BlockSpec((1, E), const),
            pl.BlockSpec((1, E), const),
            pl.BlockSpec((1, E), const),
        ],
        out_specs=(
            pl.BlockSpec((PB, N + 1, E), lambda p, b: (p * b, 0, 0)),
            pl.BlockSpec((PB, 1, E), lambda p, b: (p * b, 0, 0)),
        ),
        out_shape=(
            jax.ShapeDtypeStruct((B, N + 1, E), f32),
            jax.ShapeDtypeStruct((B, 1, E), f32),
        ),
        scratch_shapes=[
            pltpu.VMEM((B, N, E), f32),
            pltpu.VMEM((D, E), f32),
            pltpu.VMEM((N, E), jnp.bfloat16),
            pltpu.VMEM((N, E), jnp.bfloat16),
            pltpu.VMEM((1, E), f32),
            pltpu.VMEM((B, E), f32),
            pltpu.VMEM((2, E), f32),
        ],
    )(loc, locT, W_init, b_init2, W_nbr, b_nbr2, W_fin, b_fin2,
      depot2, W_dep, b_dep2, bn_w2, bn_b2)

    return (h, h_mean[:, 0, :])


# speedup vs baseline: 2.1520x; 2.1520x over previous
import functools

import jax
import jax.numpy as jnp
from jax.experimental import pallas as pl
from jax.experimental.pallas import tpu as pltpu

B, N, D, E = 16, 1000, 2, 128
K = 10
HIGH = jax.lax.Precision.HIGHEST


def _fused_kernel(loc_ref, locT_ref, w_init_ref, b_init_ref, w_nbr_ref,
                  b_nbr_ref, w_fin_ref, b_fin_ref, depot_ref, w_dep_ref,
                  b_dep_ref, bn_w_ref, bn_b_ref,
                  h_ref, hmean_ref,
                  fe_s, a_s, phi_s, plo_s, c_s, dep_s, stats_s):
    p = pl.program_id(0)
    b = pl.program_id(1)

    @pl.when(jnp.logical_and(p == 0, b == 0))
    def _prep():
        w_fin = w_fin_ref[...]
        a_s[...] = jax.lax.dot_general(
            w_init_ref[...] - 10.0 * w_nbr_ref[...], w_fin,
            (((1,), (0,)), ((), ())), precision=HIGH,
            preferred_element_type=jnp.float32)
        bw = jax.lax.dot_general(
            w_nbr_ref[...], w_fin, (((1,), (0,)), ((), ())),
            precision=HIGH, preferred_element_type=jnp.float32)
        pm = jax.lax.dot_general(
            loc_ref[0], bw, (((1,), (0,)), ((), ())),
            precision=HIGH, preferred_element_type=jnp.float32)
        phi = pm.astype(jnp.bfloat16)
        phi_s[...] = phi
        plo_s[...] = (pm - phi.astype(jnp.float32)).astype(jnp.bfloat16)
        c_s[...] = jax.lax.dot_general(
            b_init_ref[...] + 10.0 * b_nbr_ref[...], w_fin,
            (((1,), (0,)), ((), ())), precision=HIGH,
            preferred_element_type=jnp.float32) + 11.0 * b_fin_ref[...]
        dep_s[...] = jax.lax.dot_general(
            depot_ref[...], w_dep_ref[...], (((1,), (0,)), ((), ())),
            precision=HIGH,
            preferred_element_type=jnp.float32) + b_dep_ref[...]

    @pl.when(p == 0)
    def _main():
        xq = loc_ref[0]
        xqT = locT_ref[0]
        dx = xq[:, 0:1] - xqT[0:1, :]
        dy = xq[:, 1:2] - xqT[1:2, :]
        dist0 = jnp.sqrt(dx * dx + dy * dy)

        row_i = jax.lax.broadcasted_iota(jnp.int32, (N, N), 0)
        col_i = jax.lax.broadcasted_iota(jnp.int32, (N, N), 1)
        dist = jnp.where(row_i == col_i, jnp.inf, dist0)
        for _ in range(K - 1):
            m = jnp.min(dist, axis=0, keepdims=True)
            dist = jnp.where(dist == m, jnp.inf, dist)
        marked = dist == jnp.inf
        mask = marked.astype(jnp.bfloat16)
        cnt = jnp.sum(marked.astype(jnp.float32), axis=0, keepdims=True)

        def _fe_of(mk):
            return (jax.lax.dot_general(xq, a_s[...],
                                        (((1,), (0,)), ((), ())),
                                        precision=HIGH,
                                        preferred_element_type=jnp.float32)
                    + jax.lax.dot_general(mk, phi_s[...],
                                          (((0,), (0,)), ((), ())),
                                          preferred_element_type=jnp.float32)
                    + jax.lax.dot_general(mk, plo_s[...],
                                          (((0,), (0,)), ((), ())),
                                          preferred_element_type=jnp.float32)
                    + c_s[...])

        fe_s[b] = _fe_of(mask)

        @pl.when(jnp.max(cnt) > jnp.float32(K))
        def _exact_fallback():
            iota_f = row_i.astype(jnp.float32)
            big = jnp.float32(2e9)
            d2 = jnp.where(row_i == col_i, jnp.inf, dist0)
            for _ in range(K - 1):
                m2 = jnp.min(d2, axis=0, keepdims=True)
                idx = jnp.min(jnp.where(d2 == m2, iota_f, big), axis=0,
                              keepdims=True)
                d2 = jnp.where(iota_f == idx, jnp.inf, d2)
            fe_s[b] = _fe_of((d2 == jnp.inf).astype(jnp.bfloat16))

        fe = fe_s[b]
        part = jnp.stack([jnp.sum(fe, axis=0), jnp.sum(fe * fe, axis=0)])

        @pl.when(b == 0)
        def _():
            stats_s[...] = part

        @pl.when(b != 0)
        def _():
            stats_s[...] += part

    @pl.when(p == 1)
    def _finish():
        n_rows = jnp.float32(B * N)
        mean = stats_s[0:1, :] / n_rows
        var = stats_s[1:2, :] / n_rows - mean * mean
        inv = jax.lax.rsqrt(var + 1e-5)
        scale = inv * bn_w_ref[...]
        shift = bn_b_ref[...] - mean * scale

        fe = fe_s[b]
        normed = fe * scale + shift
        h = jnp.where(normed >= 0, normed, 0.01 * normed)

        dep = dep_s[pl.ds(b, 1), :]
        hdep = jnp.where(dep >= 0, dep, 0.01 * dep)
        h_ref[0, 0:1, :] = hdep
        h_ref[0, pl.ds(1, N), :] = h
        hmean_ref[0] = (hdep + jnp.sum(h, axis=0, keepdims=True)) * (
            1.0 / jnp.float32(N + 1))


@functools.partial(jax.jit, static_argnames=())
def kernel(loc, depot, W_init, b_init, W_nbr, b_nbr, W_fin, b_fin,
           W_dep, b_dep, bn_w, bn_b):
    f32 = jnp.float32
    locT = jnp.swapaxes(loc, 1, 2)
    depot2 = depot.reshape(B, 2)
    b_init2 = b_init.reshape(1, -1)
    b_nbr2 = b_nbr.reshape(1, -1)
    b_fin2 = b_fin.reshape(1, -1)
    b_dep2 = b_dep.reshape(1, -1)
    bn_w2 = bn_w.reshape(1, -1)
    bn_b2 = bn_b.reshape(1, -1)

    const = lambda p, b: (0, 0)
    h, h_mean = pl.pallas_call(
        _fused_kernel,
        grid=(2, B),
        in_specs=[
            pl.BlockSpec((1, N, D), lambda p, b: (b, 0, 0)),
            pl.BlockSpec((1, D, N), lambda p, b: (b, 0, 0)),
            pl.BlockSpec((D, 2 * E), const),
            pl.BlockSpec((1, 2 * E), const),
            pl.BlockSpec((D, 2 * E), const),
            pl.BlockSpec((1, 2 * E), const),
            pl.BlockSpec((2 * E, E), const),
            pl.BlockSpec((1, E), const),
            pl.BlockSpec((B, D), const),
            pl.BlockSpec((D, E), const),
            pl.BlockSpec((1, E), const),
            pl.BlockSpec((1, E), const),
            pl.BlockSpec((1, E), const),
        ],
        out_specs=(
            pl.BlockSpec((1, N + 1, E), lambda p, b: (p * b, 0, 0)),
            pl.BlockSpec((1, 1, E), lambda p, b: (p * b, 0, 0)),
        ),
        out_shape=(
            jax.ShapeDtypeStruct((B, N + 1, E), f32),
            jax.ShapeDtypeStruct((B, 1, E), f32),
        ),
        scratch_shapes=[
            pltpu.VMEM((B, N, E), f32),
            pltpu.VMEM((D, E), f32),
            pltpu.VMEM((N, E), jnp.bfloat16),
            pltpu.VMEM((N, E), jnp.bfloat16),
            pltpu.VMEM((1, E), f32),
            pltpu.VMEM((B, E), f32),
            pltpu.VMEM((2, E), f32),
        ],
    )(loc, locT, W_init, b_init2, W_nbr, b_nbr2, W_fin, b_fin2,
      depot2, W_dep, b_dep2, bn_w2, bn_b2)

    return (h, h_mean[:, 0, :])
